# Initial kernel scaffold; baseline (speedup 1.0000x reference)
import jax
import jax.numpy as jnp
from jax.experimental import pallas as pl


def kernel(sequence, seqment_label, wte, wse):
    raise NotImplementedError("write your pallas kernel here")


# speedup vs baseline: 1.7776x; 1.7776x over previous
import functools
import math

import jax
import jax.numpy as jnp
import numpy as np
from jax import lax
from jax.experimental import pallas as pl
from jax.experimental.pallas import tpu as pltpu
from jax.experimental.pallas import tpu_sc as plsc

_LANES = 16


def _make_pe(max_len: int, d_model: int) -> np.ndarray:
    position = np.arange(max_len, dtype=np.float32)[:, None]
    div_term = np.exp(
        np.arange(0, d_model, 2, dtype=np.float32) * (-(math.log(10000.0) / d_model))
    )
    pe = np.zeros((max_len, d_model), dtype=np.float32)
    pe[:, 0::2] = np.sin(position * div_term)
    pe[:, 1::2] = np.cos(position * div_term)
    return pe


@functools.cache
def _build_sc_kernel(N: int, D: int, V: int, C: int):
    info = plsc.get_sparse_core_info()
    NC, NS = info.num_cores, info.num_subcores
    NW = NC * NS
    assert N % NW == 0
    rows_per_w = N // NW
    CH = 64
    assert rows_per_w % CH == 0
    n_chunks = rows_per_w // CH

    mesh = plsc.VectorSubcoreMesh(core_axis_name="c", subcore_axis_name="s")

    @functools.partial(
        pl.kernel,
        mesh=mesh,
        out_type=jax.ShapeDtypeStruct((N, D), jnp.float32),
        scratch_types=[
            pltpu.VMEM((CH,), jnp.int32),
            pltpu.VMEM((CH,), jnp.int32),
            pltpu.VMEM((CH, D), jnp.float32),
            pltpu.VMEM((CH, D), jnp.float32),
            pltpu.SemaphoreType.DMA,
            pltpu.SemaphoreType.DMA,
        ],
    )
    def k(tok_hbm, cid_hbm, wte_hbm, combo_hbm, out_hbm, ti_v, ci_v, buf_a, buf_b, sem_a, sem_b):
        wid = lax.axis_index("s") * NC + lax.axis_index("c")
        base0 = wid * rows_per_w

        def chunk_body(i, carry):
            base = base0 + i * CH
            pltpu.sync_copy(tok_hbm.at[pl.ds(base, CH)], ti_v)
            pltpu.sync_copy(cid_hbm.at[pl.ds(base, CH)], ci_v)
            cp_a = pltpu.async_copy(wte_hbm.at[ti_v], buf_a, sem_a)
            cp_b = pltpu.async_copy(combo_hbm.at[ci_v], buf_b, sem_b)
            cp_a.wait()
            cp_b.wait()

            def add_row(r, c2):
                for g in range(D // _LANES):
                    sl = pl.ds(g * _LANES, _LANES)
                    buf_a[r, sl] = buf_a[r, sl] + buf_b[r, sl]
                return c2

            lax.fori_loop(0, CH, add_row, 0, unroll=False)
            pltpu.sync_copy(buf_a, out_hbm.at[pl.ds(base, CH)])
            return carry

        lax.fori_loop(0, n_chunks, chunk_body, 0, unroll=False)

    return k


def kernel(sequence, seqment_label, wte, wse):
    B, S = sequence.shape
    V, D = wte.shape
    N = B * S
    C = 3 * S

    pe = jnp.asarray(_make_pe(S, D))
    combo = (pe[:, None, :] + wse[None, :, :]).reshape(C, D)

    tok_idx = sequence.reshape(N).astype(jnp.int32)
    cid = (
        3 * jnp.arange(S, dtype=jnp.int32)[None, :]
        + seqment_label.astype(jnp.int32)
    ).reshape(N)

    k = _build_sc_kernel(N, D, V, C)
    out = k(tok_idx, cid, wte, combo)
    return out.reshape(B, S, D)


# speedup vs baseline: 2.0376x; 1.1463x over previous
import functools
import math

import jax
import jax.numpy as jnp
import numpy as np
from jax import lax
from jax.experimental import pallas as pl
from jax.experimental.pallas import tpu as pltpu
from jax.experimental.pallas import tpu_sc as plsc

_LANES = 16


def _make_pe(max_len: int, d_model: int) -> np.ndarray:
    position = np.arange(max_len, dtype=np.float32)[:, None]
    div_term = np.exp(
        np.arange(0, d_model, 2, dtype=np.float32) * (-(math.log(10000.0) / d_model))
    )
    pe = np.zeros((max_len, d_model), dtype=np.float32)
    pe[:, 0::2] = np.sin(position * div_term)
    pe[:, 1::2] = np.cos(position * div_term)
    return pe


@functools.cache
def _build_sc_kernel(N: int, D: int, V: int, C: int):
    info = plsc.get_sparse_core_info()
    NC, NS = info.num_cores, info.num_subcores
    NW = NC * NS
    assert N % NW == 0
    rows_per_w = N // NW
    CH = 40
    assert rows_per_w % (2 * CH) == 0
    n_chunks = rows_per_w // CH

    mesh = plsc.VectorSubcoreMesh(core_axis_name="c", subcore_axis_name="s")

    @functools.partial(
        pl.kernel,
        mesh=mesh,
        out_type=jax.ShapeDtypeStruct((N, D), jnp.float32),
        scratch_types=[
            pltpu.VMEM((rows_per_w,), jnp.int32),
            pltpu.VMEM((rows_per_w,), jnp.int32),
            pltpu.VMEM((CH, D), jnp.float32),
            pltpu.VMEM((CH, D), jnp.float32),
            pltpu.VMEM((CH, D), jnp.float32),
            pltpu.VMEM((CH, D), jnp.float32),
            pltpu.SemaphoreType.DMA,
            pltpu.SemaphoreType.DMA,
            pltpu.SemaphoreType.DMA,
            pltpu.SemaphoreType.DMA,
        ],
    )
    def k(tok_hbm, cid_hbm, wte_hbm, combo_hbm, out_hbm,
          ti_all, ci_all,
          buf_a0, buf_b0, buf_a1, buf_b1,
          sem_g0, sem_g1, sem_s0, sem_s1):
        wid = lax.axis_index("s") * NC + lax.axis_index("c")
        base0 = wid * rows_per_w
        bufs = ((buf_a0, buf_b0, sem_g0, sem_s0),
                (buf_a1, buf_b1, sem_g1, sem_s1))

        pltpu.sync_copy(tok_hbm.at[pl.ds(base0, rows_per_w)], ti_all)
        pltpu.sync_copy(cid_hbm.at[pl.ds(base0, rows_per_w)], ci_all)

        def start_gathers(i, slot):
            buf_a, buf_b, sem_g, _ = bufs[slot]
            ti = ti_all.at[pl.ds(i * CH, CH)]
            ci = ci_all.at[pl.ds(i * CH, CH)]
            pltpu.async_copy(wte_hbm.at[ti], buf_a, sem_g)
            pltpu.async_copy(combo_hbm.at[ci], buf_b, sem_g)

        def wait_gathers(slot):
            buf_a, buf_b, sem_g, _ = bufs[slot]
            pltpu.make_async_copy(wte_hbm.at[ti_all.at[pl.ds(0, CH)]], buf_a, sem_g).wait()
            pltpu.make_async_copy(combo_hbm.at[ci_all.at[pl.ds(0, CH)]], buf_b, sem_g).wait()

        def start_scatter(i, slot):
            buf_a, _, _, sem_s = bufs[slot]
            pltpu.async_copy(buf_a, out_hbm.at[pl.ds(base0 + i * CH, CH)], sem_s)

        def wait_scatter(slot):
            buf_a, _, _, sem_s = bufs[slot]
            pltpu.make_async_copy(buf_a, out_hbm.at[pl.ds(base0, CH)], sem_s).wait()

        def add_chunk(slot):
            buf_a, buf_b, _, _ = bufs[slot]

            def add_row(r, c2):
                for g in range(D // _LANES):
                    sl = pl.ds(g * _LANES, _LANES)
                    buf_a[r, sl] = buf_a[r, sl] + buf_b[r, sl]
                return c2

            lax.fori_loop(0, CH, add_row, 0, unroll=False)

        start_gathers(0, 0)

        def pipe_body(j, carry):
            i0 = 2 * j
            wait_gathers(0)
            add_chunk(0)
            start_scatter(i0, 0)

            @pl.when(j >= 1)
            def _():
                wait_scatter(1)

            start_gathers(i0 + 1, 1)

            wait_gathers(1)
            add_chunk(1)
            start_scatter(i0 + 1, 1)

            @pl.when(j < n_chunks // 2 - 1)
            def _():
                wait_scatter(0)
                start_gathers(i0 + 2, 0)

            return carry

        lax.fori_loop(0, n_chunks // 2, pipe_body, 0, unroll=False)
        wait_scatter(0)
        wait_scatter(1)

    return k


def kernel(sequence, seqment_label, wte, wse):
    B, S = sequence.shape
    V, D = wte.shape
    N = B * S
    C = 3 * S

    pe = jnp.asarray(_make_pe(S, D))
    combo = (pe[:, None, :] + wse[None, :, :]).reshape(C, D)

    tok_idx = sequence.reshape(N).astype(jnp.int32)
    cid = (
        3 * jnp.arange(S, dtype=jnp.int32)[None, :]
        + seqment_label.astype(jnp.int32)
    ).reshape(N)

    k = _build_sc_kernel(N, D, V, C)
    out = k(tok_idx, cid, wte, combo)
    return out.reshape(B, S, D)


# speedup vs baseline: 2.7537x; 1.3514x over previous
import functools
import math

import jax
import jax.numpy as jnp
import numpy as np
from jax import lax
from jax.experimental import pallas as pl
from jax.experimental.pallas import tpu as pltpu
from jax.experimental.pallas import tpu_sc as plsc

_LANES = 16


def _make_pe(max_len: int, d_model: int) -> np.ndarray:
    position = np.arange(max_len, dtype=np.float32)[:, None]
    div_term = np.exp(
        np.arange(0, d_model, 2, dtype=np.float32) * (-(math.log(10000.0) / d_model))
    )
    pe = np.zeros((max_len, d_model), dtype=np.float32)
    pe[:, 0::2] = np.sin(position * div_term)
    pe[:, 1::2] = np.cos(position * div_term)
    return pe


@functools.cache
def _build_sc_kernel(N: int, D: int, V: int, C: int):
    info = plsc.get_sparse_core_info()
    NC, NS = info.num_cores, info.num_subcores
    NW = NC * NS
    assert N % NW == 0
    rows_per_w = N // NW
    CH = 16
    NSLOT = 4
    assert rows_per_w % (NSLOT * CH) == 0
    n_chunks = rows_per_w // CH

    mesh = plsc.VectorSubcoreMesh(core_axis_name="c", subcore_axis_name="s")

    @functools.partial(
        pl.kernel,
        mesh=mesh,
        out_type=jax.ShapeDtypeStruct((N, D), jnp.float32),
        scratch_types=[
            pltpu.VMEM((rows_per_w,), jnp.int32),
            pltpu.VMEM((rows_per_w,), jnp.int32),
        ]
        + [pltpu.VMEM((CH, D), jnp.float32) for _ in range(NSLOT)]
        + [pltpu.VMEM((CH, D), jnp.float32) for _ in range(NSLOT)]
        + [pltpu.SemaphoreType.DMA for _ in range(2 * NSLOT)],
    )
    def k(tok_hbm, cid_hbm, wte_hbm, combo_hbm, out_hbm, *refs):
        ti_all, ci_all = refs[0], refs[1]
        bufs_a = refs[2:2 + NSLOT]
        bufs_b = refs[2 + NSLOT:2 + 2 * NSLOT]
        sems_g = refs[2 + 2 * NSLOT:2 + 3 * NSLOT]
        sems_s = refs[2 + 3 * NSLOT:2 + 4 * NSLOT]

        wid = lax.axis_index("s") * NC + lax.axis_index("c")
        base0 = wid * rows_per_w

        pltpu.sync_copy(tok_hbm.at[pl.ds(base0, rows_per_w)], ti_all)
        pltpu.sync_copy(cid_hbm.at[pl.ds(base0, rows_per_w)], ci_all)

        def start_gathers(i, slot):
            ti = ti_all.at[pl.ds(i * CH, CH)]
            ci = ci_all.at[pl.ds(i * CH, CH)]
            pltpu.async_copy(wte_hbm.at[ti], bufs_a[slot], sems_g[slot])
            pltpu.async_copy(combo_hbm.at[ci], bufs_b[slot], sems_g[slot])

        def wait_gathers(slot):
            pltpu.make_async_copy(
                wte_hbm.at[ti_all.at[pl.ds(0, CH)]], bufs_a[slot], sems_g[slot]).wait()
            pltpu.make_async_copy(
                combo_hbm.at[ci_all.at[pl.ds(0, CH)]], bufs_b[slot], sems_g[slot]).wait()

        def start_scatter(i, slot):
            pltpu.async_copy(
                bufs_a[slot], out_hbm.at[pl.ds(base0 + i * CH, CH)], sems_s[slot])

        def wait_scatter(slot):
            pltpu.make_async_copy(
                bufs_a[slot], out_hbm.at[pl.ds(base0, CH)], sems_s[slot]).wait()

        def add_chunk(slot):
            buf_a, buf_b = bufs_a[slot], bufs_b[slot]

            def add_row(r, c2):
                for g in range(D // _LANES):
                    sl = pl.ds(g * _LANES, _LANES)
                    buf_a[r, sl] = buf_a[r, sl] + buf_b[r, sl]
                return c2

            lax.fori_loop(0, CH, add_row, 0, unroll=False)

        start_gathers(0, 0)
        start_gathers(1, 1)

        def pipe_body(j, carry):
            for t in range(NSLOT):
                i = NSLOT * j + t
                wait_gathers(t)
                add_chunk(t)
                start_scatter(i, t)
                nslot = (t + 2) % NSLOT

                if t < 2:
                    @pl.when(j >= 1)
                    def _():
                        wait_scatter(nslot)
                    start_gathers(i + 2, nslot)
                else:
                    wait_scatter(nslot)

                    @pl.when(j < n_chunks // NSLOT - 1)
                    def _():
                        start_gathers(i + 2, nslot)
            return carry

        lax.fori_loop(0, n_chunks // NSLOT, pipe_body, 0, unroll=False)
        wait_scatter(2)
        wait_scatter(3)

    return k


def kernel(sequence, seqment_label, wte, wse):
    B, S = sequence.shape
    V, D = wte.shape
    N = B * S
    C = 3 * S

    pe = jnp.asarray(_make_pe(S, D))
    combo = (pe[:, None, :] + wse[None, :, :]).reshape(C, D)

    tok_idx = sequence.reshape(N).astype(jnp.int32)
    cid = (
        3 * jnp.arange(S, dtype=jnp.int32)[None, :]
        + seqment_label.astype(jnp.int32)
    ).reshape(N)

    k = _build_sc_kernel(N, D, V, C)
    out = k(tok_idx, cid, wte, combo)
    return out.reshape(B, S, D)
